# baseline (speedup 1.0000x reference)
import functools
import math

import jax
import jax.numpy as jnp
from jax import lax
from jax.experimental import pallas as pl
from jax.experimental.pallas import tpu as pltpu
from jax.experimental.pallas import tpu_sc as plsc

K = 8
N_TOTAL = 1048576
NC, NS = 2, 16
NW = NC * NS
PTS_PER_W = N_TOTAL // NW
CHUNKS_PER_W = PTS_PER_W // 128

_P5 = (0.0002749113568904233, -0.007451849578973424, 0.08095877541048518,
       -0.4594725080359905, 1.5935532480567003, -1.1936463569922442)


def _make_sc_kernel():
    mesh = plsc.VectorSubcoreMesh(core_axis_name="c", subcore_axis_name="s")

    @functools.partial(
        pl.kernel, mesh=mesh,
        out_type=jax.ShapeDtypeStruct((N_TOTAL,), jnp.float32),
        scratch_types=[
            pltpu.VMEM((2 * PTS_PER_W,), jnp.float32),
            pltpu.VMEM((PTS_PER_W,), jnp.float32),
            pltpu.VMEM((5 * K * 16,), jnp.float32),
        ],
    )
    def sc_kernel(x_hbm, coefv_hbm, out_hbm, xin, res, cf):
        wid = lax.axis_index("s") * NC + lax.axis_index("c")
        pltpu.sync_copy(coefv_hbm, cf)
        pltpu.sync_copy(x_hbm.at[pl.ds(wid * (2 * PTS_PER_W), 2 * PTS_PER_W)],
                        xin)

        def chunk_body(j, carry):
            for sub in range(8):
                off = j * 256 + sub * 16
                x0 = xin[pl.ds(off, 16)]
                x1 = xin[pl.ds(off + 128, 16)]
                x0sq = x0 * x0
                x1sq = x1 * x1
                lps = []
                for k in range(K):
                    a0 = cf[pl.ds((0 * K + k) * 16, 16)]
                    b0 = cf[pl.ds((1 * K + k) * 16, 16)]
                    a1 = cf[pl.ds((2 * K + k) * 16, 16)]
                    b1 = cf[pl.ds((3 * K + k) * 16, 16)]
                    e = cf[pl.ds((4 * K + k) * 16, 16)]
                    lps.append(a0 * x0sq + b0 * x0 + a1 * x1sq + b1 * x1 + e)
                m = lps[0]
                for k in range(1, K):
                    m = jnp.maximum(m, lps[k])
                s = jnp.exp(lps[0] - m)
                for k in range(1, K):
                    s = s + jnp.exp(lps[k] - m)
                y = jnp.full((16,), _P5[0], dtype=jnp.float32)
                for c in _P5[1:]:
                    y = y * s + c
                y = y + s * jnp.exp(-y) - 1.0
                y = y + s * jnp.exp(-y) - 1.0
                res[pl.ds(j * 128 + sub * 16, 16)] = m + y
            return carry

        lax.fori_loop(0, CHUNKS_PER_W, chunk_body, 0)
        pltpu.sync_copy(res, out_hbm.at[pl.ds(wid * PTS_PER_W, PTS_PER_W)])

    return sc_kernel


_SC_KERNEL = _make_sc_kernel()


def kernel(x, logits, means, scales):
    n = x.shape[0]
    logw = jax.nn.log_softmax(logits)
    inv2 = 1.0 / (scales * scales)
    a = -0.5 * inv2
    b = means * inv2
    e = (logw - jnp.sum(jnp.log(scales), axis=1)
         - math.log(2.0 * math.pi)
         - 0.5 * jnp.sum(means * means * inv2, axis=1))
    coef = jnp.stack([a[:, 0], b[:, 0], a[:, 1], b[:, 1], e])
    coefv = jnp.repeat(coef.reshape(-1), 16)

    pairs = n // 128
    xflat = (x.reshape(pairs, 128, 2).swapaxes(1, 2)
             .reshape(2 * pairs, 128).reshape(-1))
    return _SC_KERNEL(xflat, coefv)


# speedup vs baseline: 8.3654x; 8.3654x over previous
import functools
import math

import jax
import jax.numpy as jnp
from jax.experimental import pallas as pl
from jax.experimental.pallas import tpu as pltpu

K = 8
LANES = 128
BLOCK_PAIRS = 1024


def _body(coef_ref, x_ref, o_ref):
    x0 = x_ref[0::2, :]
    x1 = x_ref[1::2, :]

    x0sq = x0 * x0
    x1sq = x1 * x1
    lps = []
    for k in range(K):
        a0 = coef_ref[0, k]
        b0 = coef_ref[1, k]
        a1 = coef_ref[2, k]
        b1 = coef_ref[3, k]
        e = coef_ref[4, k]
        lps.append(a0 * x0sq + b0 * x0 + a1 * x1sq + b1 * x1 + e)
    m = lps[0]
    for k in range(1, K):
        m = jnp.maximum(m, lps[k])
    s = jnp.exp2(lps[0] - m)
    for k in range(1, K):
        s = s + jnp.exp2(lps[k] - m)
    o_ref[...] = (m + jnp.log2(s)) * math.log(2.0)


@functools.partial(jax.jit, static_argnames=("pairs",))
def _run(coef, x2, pairs):
    grid = pairs // BLOCK_PAIRS
    return pl.pallas_call(
        _body,
        grid=(grid,),
        in_specs=[
            pl.BlockSpec(memory_space=pltpu.SMEM),
            pl.BlockSpec((2 * BLOCK_PAIRS, LANES), lambda i: (i, 0)),
        ],
        out_specs=pl.BlockSpec((BLOCK_PAIRS, LANES), lambda i: (i, 0)),
        out_shape=jax.ShapeDtypeStruct((pairs, LANES), jnp.float32),
    )(coef, x2)


def kernel(x, logits, means, scales):
    n = x.shape[0]
    log2e = 1.0 / math.log(2.0)
    logw = jax.nn.log_softmax(logits)
    inv2 = 1.0 / (scales * scales)
    a = (-0.5 * log2e) * inv2
    b = log2e * (means * inv2)
    e = log2e * (logw - jnp.sum(jnp.log(scales), axis=1)
                 - math.log(2.0 * math.pi)
                 - 0.5 * jnp.sum(means * means * inv2, axis=1))
    coef = jnp.stack([a[:, 0], b[:, 0], a[:, 1], b[:, 1], e])

    pairs = n // LANES
    x2 = x.reshape(pairs, LANES, 2).swapaxes(1, 2).reshape(2 * pairs, LANES)
    out = _run(coef, x2, pairs)
    return out.reshape(n)
